# Initial kernel scaffold; baseline (speedup 1.0000x reference)
import jax
import jax.numpy as jnp
from jax.experimental import pallas as pl


def kernel(x, edges, weights, W1, b1, W2, b2):
    raise NotImplementedError("write your pallas kernel here")


# speedup vs baseline: 8.0620x; 8.0620x over previous
import functools

import jax
import jax.numpy as jnp
from jax import lax
from jax.experimental import pallas as pl
from jax.experimental.pallas import tpu as pltpu
from jax.experimental.pallas import tpu_sc as plsc

_NC = 2
_NS = 16
_L = 16


def _stripe(n):
    size = (n // (8 * _NS)) * 8
    last = n - size * (_NS - 1)
    return size, last


def _striped_zero_and_tail(s, n, copy_fn):
    size, last = _stripe(n)

    @pl.when(s < _NS - 1)
    def _():
        copy_fn(s * size, size)

    @pl.when(s == _NS - 1)
    def _():
        copy_fn((_NS - 1) * size, last)


def _sc_mesh():
    return plsc.VectorSubcoreMesh(
        core_axis_name="c", subcore_axis_name="s",
        num_cores=_NC, num_subcores=_NS)


def _sc_degree(dst, w2d, zeros_n1, *, n, e, batch):
    nw = _NC * _NS
    epw = e // nw
    nb = epw // batch

    @functools.partial(
        pl.kernel,
        out_type=jax.ShapeDtypeStruct((_NC * n, 1), jnp.float32),
        mesh=_sc_mesh(),
        scratch_types=[
            pltpu.VMEM((batch,), jnp.int32),
            pltpu.VMEM((batch, 1), jnp.float32),
            pltpu.VMEM_SHARED((n, 1), jnp.float32),
            pltpu.SemaphoreType.DMA,
        ],
    )
    def deg_kernel(dst_hbm, w_hbm, z_hbm, out_hbm, dst_v, w_v, acc_sh, sem):
        c = lax.axis_index("c")
        s = lax.axis_index("s")
        wid = c * _NS + s
        base = wid * epw

        def zero_stripe(r0, rows):
            pltpu.sync_copy(z_hbm.at[pl.ds(r0, rows)],
                            acc_sh.at[pl.ds(r0, rows)])

        _striped_zero_and_tail(s, n, zero_stripe)
        plsc.subcore_barrier()

        def batch_body(j, carry):
            b0 = base + j * batch
            pltpu.sync_copy(dst_hbm.at[pl.ds(b0, batch)], dst_v)
            pltpu.sync_copy(w_hbm.at[pl.ds(b0, batch)], w_v)
            pltpu.sync_copy(w_v, acc_sh.at[dst_v], add=True)
            return carry

        lax.fori_loop(0, nb, batch_body, 0)
        plsc.subcore_barrier()

        def write_stripe(r0, rows):
            pltpu.sync_copy(acc_sh.at[pl.ds(r0, rows)],
                            out_hbm.at[pl.ds(c * n + r0, rows)])

        _striped_zero_and_tail(s, n, write_stripe)

    return deg_kernel(dst, w2d, zeros_n1)


def _sc_aggregate(hs, src, dst, w, zeros_nd, *, n, d, e, batch):
    nw = _NC * _NS
    epw = e // nw
    nb = epw // batch

    @functools.partial(
        pl.kernel,
        out_type=jax.ShapeDtypeStruct((_NC * n, d), jnp.float32),
        mesh=_sc_mesh(),
        scratch_types=[
            pltpu.VMEM((batch,), jnp.int32),
            pltpu.VMEM((batch,), jnp.int32),
            pltpu.VMEM((batch,), jnp.float32),
            pltpu.VMEM((batch, d), jnp.float32),
            pltpu.VMEM_SHARED((n, d), jnp.float32),
            pltpu.SemaphoreType.DMA,
        ],
    )
    def agg_kernel(hs_hbm, src_hbm, dst_hbm, w_hbm, z_hbm, out_hbm,
                   src_v, dst_v, w_v, rows_v, acc_sh, sem):
        c = lax.axis_index("c")
        s = lax.axis_index("s")
        wid = c * _NS + s
        base = wid * epw

        def zero_stripe(r0, rows):
            pltpu.sync_copy(z_hbm.at[pl.ds(r0, rows)],
                            acc_sh.at[pl.ds(r0, rows)])

        _striped_zero_and_tail(s, n, zero_stripe)
        plsc.subcore_barrier()

        def batch_body(j, carry):
            b0 = base + j * batch
            pltpu.sync_copy(src_hbm.at[pl.ds(b0, batch)], src_v)
            pltpu.sync_copy(dst_hbm.at[pl.ds(b0, batch)], dst_v)
            pltpu.sync_copy(w_hbm.at[pl.ds(b0, batch)], w_v)
            pltpu.async_copy(hs_hbm.at[src_v], rows_v, sem).wait()

            def group_body(g, carry2):
                w_vec = w_v[pl.ds(g * _L, _L)]
                for i in range(_L):
                    we = w_vec[i]
                    row = g * _L + i
                    for cc in range(d // _L):
                        sl = pl.ds(cc * _L, _L)
                        rows_v[row, sl] = rows_v[row, sl] * we
                return carry2

            lax.fori_loop(0, batch // _L, group_body, 0)
            pltpu.sync_copy(rows_v, acc_sh.at[dst_v], add=True)
            return carry

        lax.fori_loop(0, nb, batch_body, 0)
        plsc.subcore_barrier()

        def write_stripe(r0, rows):
            pltpu.sync_copy(acc_sh.at[pl.ds(r0, rows)],
                            out_hbm.at[pl.ds(c * n + r0, rows)])

        _striped_zero_and_tail(s, n, write_stripe)

    return agg_kernel(hs, src, dst, w, zeros_nd)


def _row_spec(bn, d, offset_blocks=0):
    return pl.BlockSpec((bn, d), lambda i, o=offset_blocks: (i + o, 0))


def _tc_layer1(x, w1, degp, *, n, d, bn):
    grid = (n // bn,)

    def body(x_ref, w_ref, d0_ref, d1_ref, out_ref):
        deg = d0_ref[...] + d1_ref[...] + 2.0
        dis = lax.rsqrt(deg)
        h = jnp.dot(x_ref[...], w_ref[...],
                    preferred_element_type=jnp.float32)
        out_ref[...] = h * dis

    return pl.pallas_call(
        body,
        grid=grid,
        in_specs=[
            _row_spec(bn, d),
            pl.BlockSpec((d, d), lambda i: (0, 0)),
            _row_spec(bn, 1),
            _row_spec(bn, 1, n // bn),
        ],
        out_specs=_row_spec(bn, d),
        out_shape=jax.ShapeDtypeStruct((n, d), jnp.float32),
    )(x, w1, degp, degp)


def _tc_layer2(aggp, hs, degp, b1, w2, *, n, d, bn):
    grid = (n // bn,)

    def body(a0_ref, a1_ref, hs_ref, d0_ref, d1_ref, b_ref, w_ref, out_ref):
        deg = d0_ref[...] + d1_ref[...] + 2.0
        dis = lax.rsqrt(deg)
        t = dis * (a0_ref[...] + a1_ref[...]) + (2.0 * dis) * hs_ref[...]
        z = jnp.maximum(t + b_ref[...], 0.0)
        h = jnp.dot(z, w_ref[...], preferred_element_type=jnp.float32)
        out_ref[...] = h * dis

    return pl.pallas_call(
        body,
        grid=grid,
        in_specs=[
            _row_spec(bn, d),
            _row_spec(bn, d, n // bn),
            _row_spec(bn, d),
            _row_spec(bn, 1),
            _row_spec(bn, 1, n // bn),
            pl.BlockSpec((1, d), lambda i: (0, 0)),
            pl.BlockSpec((d, d), lambda i: (0, 0)),
        ],
        out_specs=_row_spec(bn, d),
        out_shape=jax.ShapeDtypeStruct((n, d), jnp.float32),
    )(aggp, aggp, hs, degp, degp, b1, w2)


def _tc_final(aggp, hs, degp, b2, *, n, d, bn):
    grid = (n // bn,)

    def body(a0_ref, a1_ref, hs_ref, d0_ref, d1_ref, b_ref, out_ref):
        deg = d0_ref[...] + d1_ref[...] + 2.0
        dis = lax.rsqrt(deg)
        t = dis * (a0_ref[...] + a1_ref[...]) + (2.0 * dis) * hs_ref[...]
        out_ref[...] = jax.nn.sigmoid(t + b_ref[...])

    return pl.pallas_call(
        body,
        grid=grid,
        in_specs=[
            _row_spec(bn, d),
            _row_spec(bn, d, n // bn),
            _row_spec(bn, d),
            _row_spec(bn, 1),
            _row_spec(bn, 1, n // bn),
            pl.BlockSpec((1, d), lambda i: (0, 0)),
        ],
        out_specs=_row_spec(bn, d),
        out_shape=jax.ShapeDtypeStruct((n, d), jnp.float32),
    )(aggp, aggp, hs, degp, degp, b2)


def kernel(x, edges, weights, W1, b1, W2, b2):
    n, d = x.shape
    e = edges.shape[1]
    batch = 80
    bn = 1000

    src = edges[0].astype(jnp.int32)
    dst = edges[1].astype(jnp.int32)
    w = weights.astype(jnp.float32)
    w2d = w.reshape(e, 1)
    b1r = b1.reshape(1, d)
    b2r = b2.reshape(1, d)
    zeros_n1 = jnp.zeros((n, 1), jnp.float32)
    zeros_nd = jnp.zeros((n, d), jnp.float32)

    degp = _sc_degree(dst, w2d, zeros_n1, n=n, e=e, batch=batch)
    h1s = _tc_layer1(x, W1, degp, n=n, d=d, bn=bn)
    agg1 = _sc_aggregate(h1s, src, dst, w, zeros_nd, n=n, d=d, e=e,
                         batch=batch)
    h2s = _tc_layer2(agg1, h1s, degp, b1r, W2, n=n, d=d, bn=bn)
    agg2 = _sc_aggregate(h2s, src, dst, w, zeros_nd, n=n, d=d, e=e,
                         batch=batch)
    return _tc_final(agg2, h2s, degp, b2r, n=n, d=d, bn=bn)


# speedup vs baseline: 20.3081x; 2.5190x over previous
import functools

import jax
import jax.numpy as jnp
from jax import lax
from jax.experimental import pallas as pl
from jax.experimental.pallas import tpu as pltpu
from jax.experimental.pallas import tpu_sc as plsc

_NC = 2
_NS = 16
_L = 16
_NW = _NC * _NS


def _sc_mesh():
    return plsc.VectorSubcoreMesh(
        core_axis_name="c", subcore_axis_name="s",
        num_cores=_NC, num_subcores=_NS)


def _stripe(n):
    size = (n // (8 * _NS)) * 8
    last = n - size * (_NS - 1)
    return size, last


def _per_stripe(s, n, fn):
    size, last = _stripe(n)

    @pl.when(s < _NS - 1)
    def _():
        fn(s * size, size)

    @pl.when(s == _NS - 1)
    def _():
        fn((_NS - 1) * size, last)


def _sc_degree(dst, w, *, n, e, batch):
    epw = e // _NW
    nb = epw // batch
    dw = _L

    @functools.partial(
        pl.kernel,
        out_type=jax.ShapeDtypeStruct((_NC * n, dw), jnp.float32),
        mesh=_sc_mesh(),
        scratch_types=[
            pltpu.VMEM((batch,), jnp.int32),
            pltpu.VMEM((batch,), jnp.float32),
            pltpu.VMEM((batch, dw), jnp.float32),
            pltpu.VMEM_SHARED((n, dw), jnp.float32),
        ],
        name="sc_gcn_degree",
    )
    def deg_kernel(dst_hbm, w_hbm, out_hbm, db, wb, vals, acc_sh):
        c = lax.axis_index("c")
        s = lax.axis_index("s")
        wid = c * _NS + s
        base = wid * epw

        def zrow(i, carry):
            vals[i, pl.ds(0, _L)] = jnp.zeros((_L,), jnp.float32)
            return carry

        lax.fori_loop(0, batch, zrow, 0)

        def zero_stripe(r0, rows):
            nfull, tail = rows // batch, rows % batch
            for k in range(nfull):
                pltpu.sync_copy(vals,
                                acc_sh.at[pl.ds(r0 + k * batch, batch)])
            if tail:
                pltpu.sync_copy(vals.at[pl.ds(0, tail)],
                                acc_sh.at[pl.ds(r0 + nfull * batch, tail)])

        _per_stripe(s, n, zero_stripe)
        plsc.subcore_barrier()

        def batch_body(j, carry):
            b0 = base + j * batch
            pltpu.sync_copy(dst_hbm.at[pl.ds(b0, batch)], db)
            pltpu.sync_copy(w_hbm.at[pl.ds(b0, batch)], wb)

            def group_body(g, carry2):
                w_vec = wb[pl.ds(g * _L, _L)]
                for i in range(_L):
                    we = w_vec[i]
                    vals[g * _L + i, pl.ds(0, _L)] = (
                        jnp.ones((_L,), jnp.float32) * we)
                return carry2

            lax.fori_loop(0, batch // _L, group_body, 0)
            pltpu.sync_copy(vals, acc_sh.at[db], add=True)
            return carry

        lax.fori_loop(0, nb, batch_body, 0)
        plsc.subcore_barrier()

        def write_stripe(r0, rows):
            pltpu.sync_copy(acc_sh.at[pl.ds(r0, rows)],
                            out_hbm.at[pl.ds(c * n + r0, rows)])

        _per_stripe(s, n, write_stripe)

    return deg_kernel(dst, w)


def _sc_aggregate(hs, src, dst, w, *, n, d, e, batch):
    epw = e // _NW
    nb = epw // batch

    @functools.partial(
        pl.kernel,
        out_type=jax.ShapeDtypeStruct((_NC * n, d), jnp.float32),
        mesh=_sc_mesh(),
        scratch_types=[
            pltpu.VMEM((epw,), jnp.int32),
            pltpu.VMEM((epw,), jnp.float32),
            pltpu.VMEM((batch,), jnp.int32),
            pltpu.VMEM((batch,), jnp.int32),
            pltpu.VMEM((batch, d), jnp.float32),
            pltpu.VMEM((batch, d), jnp.float32),
            pltpu.VMEM_SHARED((n, d), jnp.float32),
            pltpu.SemaphoreType.DMA,
            pltpu.SemaphoreType.DMA,
            pltpu.SemaphoreType.DMA,
            pltpu.SemaphoreType.DMA,
        ],
        name="sc_gcn_aggregate",
    )
    def agg_kernel(hs_hbm, src_hbm, dst_hbm, w_hbm, out_hbm,
                   src_v, w_v, db0, db1, rows0, rows1, acc_sh,
                   gs0, gs1, dsem0, dsem1):
        c = lax.axis_index("c")
        s = lax.axis_index("s")
        wid = c * _NS + s
        base = wid * epw
        pltpu.sync_copy(src_hbm.at[pl.ds(base, epw)], src_v)
        pltpu.sync_copy(w_hbm.at[pl.ds(base, epw)], w_v)

        def zrow(i, carry):
            for cc in range(d // _L):
                rows0[i, pl.ds(cc * _L, _L)] = jnp.zeros((_L,), jnp.float32)
            return carry

        lax.fori_loop(0, batch, zrow, 0)

        def zero_stripe(r0, rows):
            nfull, tail = rows // batch, rows % batch
            for k in range(nfull):
                pltpu.sync_copy(rows0,
                                acc_sh.at[pl.ds(r0 + k * batch, batch)])
            if tail:
                pltpu.sync_copy(rows0.at[pl.ds(0, tail)],
                                acc_sh.at[pl.ds(r0 + nfull * batch, tail)])

        _per_stripe(s, n, zero_stripe)
        plsc.subcore_barrier()

        def gather(j, buf, sem):
            pltpu.async_copy(hs_hbm.at[src_v.at[pl.ds(j * batch, batch)]],
                             buf, sem)

        def wait_gather(buf, sem):
            pltpu.make_async_copy(hs_hbm.at[src_v.at[pl.ds(0, batch)]],
                                  buf, sem).wait()

        def dload(j, db, dsem):
            pltpu.async_copy(dst_hbm.at[pl.ds(base + j * batch, batch)],
                             db, dsem)

        def wait_dload(db, dsem):
            pltpu.make_async_copy(dst_hbm.at[pl.ds(base, batch)],
                                  db, dsem).wait()

        def scale(j, buf):
            def group_body(g, carry):
                w_vec = w_v[pl.ds(j * batch + g * _L, _L)]
                for i in range(_L):
                    we = w_vec[i]
                    row = g * _L + i
                    for cc in range(d // _L):
                        sl = pl.ds(cc * _L, _L)
                        buf[row, sl] = buf[row, sl] * we
                return carry

            lax.fori_loop(0, batch // _L, group_body, 0)

        def scatter(buf, db):
            pltpu.sync_copy(buf, acc_sh.at[db], add=True)

        dload(0, db0, dsem0)
        gather(0, rows0, gs0)
        dload(1, db1, dsem1)

        def body(jj, carry):
            j0 = 2 * jj
            gather(j0 + 1, rows1, gs1)
            wait_gather(rows0, gs0)
            scale(j0, rows0)
            wait_dload(db0, dsem0)
            scatter(rows0, db0)
            dload(j0 + 2, db0, dsem0)
            gather(j0 + 2, rows0, gs0)
            wait_gather(rows1, gs1)
            scale(j0 + 1, rows1)
            wait_dload(db1, dsem1)
            scatter(rows1, db1)

            @pl.when(j0 + 3 < nb)
            def _():
                dload(j0 + 3, db1, dsem1)

            return carry

        lax.fori_loop(0, (nb - 1) // 2, body, 0)
        wait_gather(rows0, gs0)
        scale(nb - 1, rows0)
        wait_dload(db0, dsem0)
        scatter(rows0, db0)
        plsc.subcore_barrier()

        def write_stripe(r0, rows):
            pltpu.sync_copy(acc_sh.at[pl.ds(r0, rows)],
                            out_hbm.at[pl.ds(c * n + r0, rows)])

        _per_stripe(s, n, write_stripe)

    return agg_kernel(hs, src, dst, w)


def _row_spec(bn, d, offset_blocks=0):
    return pl.BlockSpec((bn, d), lambda i, o=offset_blocks: (i + o, 0))


def _tc_layer1(x, w1, degp, *, n, d, bn):
    grid = (n // bn,)

    def body(x_ref, w_ref, d0_ref, d1_ref, out_ref):
        deg = d0_ref[:, 0:1] + d1_ref[:, 0:1] + 2.0
        dis = lax.rsqrt(deg)
        h = jnp.dot(x_ref[...], w_ref[...],
                    preferred_element_type=jnp.float32)
        out_ref[...] = h * dis

    return pl.pallas_call(
        body,
        grid=grid,
        in_specs=[
            _row_spec(bn, d),
            pl.BlockSpec((d, d), lambda i: (0, 0)),
            _row_spec(bn, 16),
            _row_spec(bn, 16, n // bn),
        ],
        out_specs=_row_spec(bn, d),
        out_shape=jax.ShapeDtypeStruct((n, d), jnp.float32),
    )(x, w1, degp, degp)


def _tc_layer2(aggp, hs, degp, b1, w2, *, n, d, bn):
    grid = (n // bn,)

    def body(a0_ref, a1_ref, hs_ref, d0_ref, d1_ref, b_ref, w_ref, out_ref):
        deg = d0_ref[:, 0:1] + d1_ref[:, 0:1] + 2.0
        dis = lax.rsqrt(deg)
        t = dis * (a0_ref[...] + a1_ref[...]) + (2.0 * dis) * hs_ref[...]
        z = jnp.maximum(t + b_ref[...], 0.0)
        h = jnp.dot(z, w_ref[...], preferred_element_type=jnp.float32)
        out_ref[...] = h * dis

    return pl.pallas_call(
        body,
        grid=grid,
        in_specs=[
            _row_spec(bn, d),
            _row_spec(bn, d, n // bn),
            _row_spec(bn, d),
            _row_spec(bn, 16),
            _row_spec(bn, 16, n // bn),
            pl.BlockSpec((1, d), lambda i: (0, 0)),
            pl.BlockSpec((d, d), lambda i: (0, 0)),
        ],
        out_specs=_row_spec(bn, d),
        out_shape=jax.ShapeDtypeStruct((n, d), jnp.float32),
    )(aggp, aggp, hs, degp, degp, b1, w2)


def _tc_final(aggp, hs, degp, b2, *, n, d, bn):
    grid = (n // bn,)

    def body(a0_ref, a1_ref, hs_ref, d0_ref, d1_ref, b_ref, out_ref):
        deg = d0_ref[:, 0:1] + d1_ref[:, 0:1] + 2.0
        dis = lax.rsqrt(deg)
        t = dis * (a0_ref[...] + a1_ref[...]) + (2.0 * dis) * hs_ref[...]
        out_ref[...] = jax.nn.sigmoid(t + b_ref[...])

    return pl.pallas_call(
        body,
        grid=grid,
        in_specs=[
            _row_spec(bn, d),
            _row_spec(bn, d, n // bn),
            _row_spec(bn, d),
            _row_spec(bn, 16),
            _row_spec(bn, 16, n // bn),
            pl.BlockSpec((1, d), lambda i: (0, 0)),
        ],
        out_specs=_row_spec(bn, d),
        out_shape=jax.ShapeDtypeStruct((n, d), jnp.float32),
    )(aggp, aggp, hs, degp, degp, b2)


def kernel(x, edges, weights, W1, b1, W2, b2):
    n, d = x.shape
    e = edges.shape[1]
    epw = e // _NW
    batch_a = 80
    batch_d = 80
    bn = 1000

    src = edges[0].astype(jnp.int32)
    dst = edges[1].astype(jnp.int32)
    w = weights.astype(jnp.float32)
    b1r = b1.reshape(1, d)
    b2r = b2.reshape(1, d)

    degp = _sc_degree(dst, w, n=n, e=e, batch=batch_d)
    h1s = _tc_layer1(x, W1, degp, n=n, d=d, bn=bn)
    agg1 = _sc_aggregate(h1s, src, dst, w, n=n, d=d, e=e, batch=batch_a)
    h2s = _tc_layer2(agg1, h1s, degp, b1r, W2, n=n, d=d, bn=bn)
    agg2 = _sc_aggregate(h2s, src, dst, w, n=n, d=d, e=e, batch=batch_a)
    return _tc_final(agg2, h2s, degp, b2r, n=n, d=d, bn=bn)


# speedup vs baseline: 24.7593x; 1.2192x over previous
import functools

import jax
import jax.numpy as jnp
from jax import lax
from jax.experimental import pallas as pl
from jax.experimental.pallas import tpu as pltpu
from jax.experimental.pallas import tpu_sc as plsc

_NC = 2
_NS = 16
_L = 16
_NW = _NC * _NS


def _sc_mesh():
    return plsc.VectorSubcoreMesh(
        core_axis_name="c", subcore_axis_name="s",
        num_cores=_NC, num_subcores=_NS)


def _stripe(n):
    size = (n // (8 * _NS)) * 8
    last = n - size * (_NS - 1)
    return size, last


def _per_stripe(s, n, fn):
    size, last = _stripe(n)

    @pl.when(s < _NS - 1)
    def _():
        fn(s * size, size)

    @pl.when(s == _NS - 1)
    def _():
        fn((_NS - 1) * size, last)


def _pipeline2(nb, load, wait_load, work, scat, wait_scat):
    assert nb % 2 == 1
    load(0, 0)

    def body(jj, carry):
        j0 = 2 * jj

        @pl.when(jj > 0)
        def _():
            wait_scat(1)

        load(j0 + 1, 1)
        wait_load(0)
        work(j0, 0)
        scat(j0, 0)
        wait_scat(0)
        load(j0 + 2, 0)
        wait_load(1)
        work(j0 + 1, 1)
        scat(j0 + 1, 1)
        return carry

    lax.fori_loop(0, (nb - 1) // 2, body, 0)
    wait_load(0)
    work(nb - 1, 0)
    scat(nb - 1, 0)
    wait_scat(1)
    wait_scat(0)


def _sc_degree(dst, w, *, n, e, batch):
    epw = e // _NW
    nb = epw // batch
    dw = _L

    @functools.partial(
        pl.kernel,
        out_type=jax.ShapeDtypeStruct((_NC * n, dw), jnp.float32),
        mesh=_sc_mesh(),
        scratch_types=[
            [pltpu.VMEM((batch,), jnp.int32)] * 2,
            [pltpu.VMEM((batch,), jnp.float32)] * 2,
            [pltpu.VMEM((batch, dw), jnp.float32)] * 2,
            pltpu.VMEM_SHARED((n, dw), jnp.float32),
            [pltpu.SemaphoreType.DMA] * 2,
            [pltpu.SemaphoreType.DMA] * 2,
            [pltpu.SemaphoreType.DMA] * 2,
        ],
        name="sc_gcn_degree",
    )
    def deg_kernel(dst_hbm, w_hbm, out_hbm, db, wb, vals, acc_sh,
                   dsem, wsem, scsem):
        c = lax.axis_index("c")
        s = lax.axis_index("s")
        wid = c * _NS + s
        base = wid * epw

        def zrow(i, carry):
            vals[0][i, pl.ds(0, _L)] = jnp.zeros((_L,), jnp.float32)
            return carry

        lax.fori_loop(0, batch, zrow, 0)

        def zero_stripe(r0, nrows):
            nfull, tail = nrows // batch, nrows % batch
            for k in range(nfull):
                pltpu.sync_copy(vals[0],
                                acc_sh.at[pl.ds(r0 + k * batch, batch)])
            if tail:
                pltpu.sync_copy(vals[0].at[pl.ds(0, tail)],
                                acc_sh.at[pl.ds(r0 + nfull * batch, tail)])

        _per_stripe(s, n, zero_stripe)
        plsc.subcore_barrier()

        def load(j, p):
            b0 = base + j * batch
            pltpu.async_copy(dst_hbm.at[pl.ds(b0, batch)], db[p], dsem[p])
            pltpu.async_copy(w_hbm.at[pl.ds(b0, batch)], wb[p], wsem[p])

        def wait_load(p):
            pltpu.make_async_copy(dst_hbm.at[pl.ds(base, batch)],
                                  db[p], dsem[p]).wait()
            pltpu.make_async_copy(w_hbm.at[pl.ds(base, batch)],
                                  wb[p], wsem[p]).wait()

        def work(j, p):
            def group_body(g, carry2):
                w_vec = wb[p][pl.ds(g * _L, _L)]
                for i in range(_L):
                    we = w_vec[i]
                    vals[p][g * _L + i, pl.ds(0, _L)] = (
                        jnp.ones((_L,), jnp.float32) * we)
                return carry2

            lax.fori_loop(0, batch // _L, group_body, 0)

        def scat(j, p):
            pltpu.async_copy(vals[p], acc_sh.at[db[p]], scsem[p], add=True)

        def wait_scat(p):
            pltpu.make_async_copy(vals[p], acc_sh.at[db[p]],
                                  scsem[p]).wait()

        _pipeline2(nb, load, wait_load, work, scat, wait_scat)
        plsc.subcore_barrier()

        def write_stripe(r0, rows):
            pltpu.sync_copy(acc_sh.at[pl.ds(r0, rows)],
                            out_hbm.at[pl.ds(c * n + r0, rows)])

        _per_stripe(s, n, write_stripe)

    return deg_kernel(dst, w)


def _sc_aggregate(hs, src, dst, w, *, n, d, e, batch):
    epw = e // _NW
    nb = epw // batch

    @functools.partial(
        pl.kernel,
        out_type=jax.ShapeDtypeStruct((_NC * n, d), jnp.float32),
        mesh=_sc_mesh(),
        scratch_types=[
            pltpu.VMEM((epw,), jnp.int32),
            pltpu.VMEM((epw,), jnp.float32),
            [pltpu.VMEM((batch,), jnp.int32)] * 2,
            [pltpu.VMEM((batch, d), jnp.float32)] * 2,
            pltpu.VMEM_SHARED((n, d), jnp.float32),
            [pltpu.SemaphoreType.DMA] * 2,
            [pltpu.SemaphoreType.DMA] * 2,
            [pltpu.SemaphoreType.DMA] * 2,
        ],
        name="sc_gcn_aggregate",
    )
    def agg_kernel(hs_hbm, src_hbm, dst_hbm, w_hbm, out_hbm,
                   src_v, w_v, db, rows, acc_sh, gsem, dsem, scsem):
        c = lax.axis_index("c")
        s = lax.axis_index("s")
        wid = c * _NS + s
        base = wid * epw
        pltpu.sync_copy(src_hbm.at[pl.ds(base, epw)], src_v)
        pltpu.sync_copy(w_hbm.at[pl.ds(base, epw)], w_v)

        def zrow(i, carry):
            for cc in range(d // _L):
                rows[0][i, pl.ds(cc * _L, _L)] = jnp.zeros((_L,), jnp.float32)
            return carry

        lax.fori_loop(0, batch, zrow, 0)

        def zero_stripe(r0, nrows):
            nfull, tail = nrows // batch, nrows % batch
            for k in range(nfull):
                pltpu.sync_copy(rows[0],
                                acc_sh.at[pl.ds(r0 + k * batch, batch)])
            if tail:
                pltpu.sync_copy(rows[0].at[pl.ds(0, tail)],
                                acc_sh.at[pl.ds(r0 + nfull * batch, tail)])

        _per_stripe(s, n, zero_stripe)
        plsc.subcore_barrier()

        def load(j, p):
            pltpu.async_copy(dst_hbm.at[pl.ds(base + j * batch, batch)],
                             db[p], dsem[p])
            pltpu.async_copy(hs_hbm.at[src_v.at[pl.ds(j * batch, batch)]],
                             rows[p], gsem[p])

        def wait_load(p):
            pltpu.make_async_copy(hs_hbm.at[src_v.at[pl.ds(0, batch)]],
                                  rows[p], gsem[p]).wait()

        def work(j, p):
            buf = rows[p]

            def group_body(g, carry):
                w_vec = w_v[pl.ds(j * batch + g * _L, _L)]
                for i in range(_L):
                    we = w_vec[i]
                    row = g * _L + i
                    for cc in range(d // _L):
                        sl = pl.ds(cc * _L, _L)
                        buf[row, sl] = buf[row, sl] * we
                return carry

            lax.fori_loop(0, batch // _L, group_body, 0)

        def scat(j, p):
            pltpu.make_async_copy(dst_hbm.at[pl.ds(base, batch)],
                                  db[p], dsem[p]).wait()
            pltpu.async_copy(rows[p], acc_sh.at[db[p]], scsem[p], add=True)

        def wait_scat(p):
            pltpu.make_async_copy(rows[p], acc_sh.at[db[p]],
                                  scsem[p]).wait()

        _pipeline2(nb, load, wait_load, work, scat, wait_scat)
        plsc.subcore_barrier()

        def write_stripe(r0, rows):
            pltpu.sync_copy(acc_sh.at[pl.ds(r0, rows)],
                            out_hbm.at[pl.ds(c * n + r0, rows)])

        _per_stripe(s, n, write_stripe)

    return agg_kernel(hs, src, dst, w)


def _row_spec(bn, d, offset_blocks=0):
    return pl.BlockSpec((bn, d), lambda i, o=offset_blocks: (i + o, 0))


def _tc_layer1(x, w1, degp, *, n, d, bn):
    grid = (n // bn,)

    def body(x_ref, w_ref, d0_ref, d1_ref, out_ref):
        deg = d0_ref[:, 0:1] + d1_ref[:, 0:1] + 2.0
        dis = lax.rsqrt(deg)
        h = jnp.dot(x_ref[...], w_ref[...],
                    preferred_element_type=jnp.float32)
        out_ref[...] = h * dis

    return pl.pallas_call(
        body,
        grid=grid,
        in_specs=[
            _row_spec(bn, d),
            pl.BlockSpec((d, d), lambda i: (0, 0)),
            _row_spec(bn, 16),
            _row_spec(bn, 16, n // bn),
        ],
        out_specs=_row_spec(bn, d),
        out_shape=jax.ShapeDtypeStruct((n, d), jnp.float32),
    )(x, w1, degp, degp)


def _tc_layer2(aggp, hs, degp, b1, w2, *, n, d, bn):
    grid = (n // bn,)

    def body(a0_ref, a1_ref, hs_ref, d0_ref, d1_ref, b_ref, w_ref, out_ref):
        deg = d0_ref[:, 0:1] + d1_ref[:, 0:1] + 2.0
        dis = lax.rsqrt(deg)
        t = dis * (a0_ref[...] + a1_ref[...]) + (2.0 * dis) * hs_ref[...]
        z = jnp.maximum(t + b_ref[...], 0.0)
        h = jnp.dot(z, w_ref[...], preferred_element_type=jnp.float32)
        out_ref[...] = h * dis

    return pl.pallas_call(
        body,
        grid=grid,
        in_specs=[
            _row_spec(bn, d),
            _row_spec(bn, d, n // bn),
            _row_spec(bn, d),
            _row_spec(bn, 16),
            _row_spec(bn, 16, n // bn),
            pl.BlockSpec((1, d), lambda i: (0, 0)),
            pl.BlockSpec((d, d), lambda i: (0, 0)),
        ],
        out_specs=_row_spec(bn, d),
        out_shape=jax.ShapeDtypeStruct((n, d), jnp.float32),
    )(aggp, aggp, hs, degp, degp, b1, w2)


def _tc_final(aggp, hs, degp, b2, *, n, d, bn):
    grid = (n // bn,)

    def body(a0_ref, a1_ref, hs_ref, d0_ref, d1_ref, b_ref, out_ref):
        deg = d0_ref[:, 0:1] + d1_ref[:, 0:1] + 2.0
        dis = lax.rsqrt(deg)
        t = dis * (a0_ref[...] + a1_ref[...]) + (2.0 * dis) * hs_ref[...]
        out_ref[...] = jax.nn.sigmoid(t + b_ref[...])

    return pl.pallas_call(
        body,
        grid=grid,
        in_specs=[
            _row_spec(bn, d),
            _row_spec(bn, d, n // bn),
            _row_spec(bn, d),
            _row_spec(bn, 16),
            _row_spec(bn, 16, n // bn),
            pl.BlockSpec((1, d), lambda i: (0, 0)),
        ],
        out_specs=_row_spec(bn, d),
        out_shape=jax.ShapeDtypeStruct((n, d), jnp.float32),
    )(aggp, aggp, hs, degp, degp, b2)


def kernel(x, edges, weights, W1, b1, W2, b2):
    n, d = x.shape
    e = edges.shape[1]
    epw = e // _NW
    batch_a = 80
    batch_d = 80
    bn = 1000

    src = edges[0].astype(jnp.int32)
    dst = edges[1].astype(jnp.int32)
    w = weights.astype(jnp.float32)
    b1r = b1.reshape(1, d)
    b2r = b2.reshape(1, d)

    degp = _sc_degree(dst, w, n=n, e=e, batch=batch_d)
    h1s = _tc_layer1(x, W1, degp, n=n, d=d, bn=bn)
    agg1 = _sc_aggregate(h1s, src, dst, w, n=n, d=d, e=e, batch=batch_a)
    h2s = _tc_layer2(agg1, h1s, degp, b1r, W2, n=n, d=d, bn=bn)
    agg2 = _sc_aggregate(h2s, src, dst, w, n=n, d=d, e=e, batch=batch_a)
    return _tc_final(agg2, h2s, degp, b2r, n=n, d=d, bn=bn)


# speedup vs baseline: 25.2439x; 1.0196x over previous
import functools

import jax
import jax.numpy as jnp
from jax import lax
from jax.experimental import pallas as pl
from jax.experimental.pallas import tpu as pltpu
from jax.experimental.pallas import tpu_sc as plsc

_NC = 2
_NS = 16
_L = 16
_NW = _NC * _NS


def _sc_mesh():
    return plsc.VectorSubcoreMesh(
        core_axis_name="c", subcore_axis_name="s",
        num_cores=_NC, num_subcores=_NS)


def _stripe(n):
    size = (n // (8 * _NS)) * 8
    last = n - size * (_NS - 1)
    return size, last


def _per_stripe(s, n, fn):
    size, last = _stripe(n)

    @pl.when(s < _NS - 1)
    def _():
        fn(s * size, size)

    @pl.when(s == _NS - 1)
    def _():
        fn((_NS - 1) * size, last)


def _pipeline2(nb, load, wait_load, work, scat, wait_scat):
    assert nb % 2 == 1
    load(0, 0)

    def body(jj, carry):
        j0 = 2 * jj

        @pl.when(jj > 0)
        def _():
            wait_scat(1)

        load(j0 + 1, 1)
        wait_load(0)
        work(j0, 0)
        scat(j0, 0)
        wait_scat(0)
        load(j0 + 2, 0)
        wait_load(1)
        work(j0 + 1, 1)
        scat(j0 + 1, 1)
        return carry

    lax.fori_loop(0, (nb - 1) // 2, body, 0)
    wait_load(0)
    work(nb - 1, 0)
    scat(nb - 1, 0)
    wait_scat(1)
    wait_scat(0)


def _sc_degree(dst, w, *, n, e, batch):
    epw = e // _NW
    nb = epw // batch
    dw = _L

    @functools.partial(
        pl.kernel,
        out_type=jax.ShapeDtypeStruct((_NC * n, dw), jnp.float32),
        mesh=_sc_mesh(),
        scratch_types=[
            [pltpu.VMEM((batch,), jnp.int32)] * 2,
            [pltpu.VMEM((batch,), jnp.float32)] * 2,
            [pltpu.VMEM((batch, dw), jnp.float32)] * 2,
            pltpu.VMEM_SHARED((n, dw), jnp.float32),
            [pltpu.SemaphoreType.DMA] * 2,
            [pltpu.SemaphoreType.DMA] * 2,
            [pltpu.SemaphoreType.DMA] * 2,
        ],
        name="sc_gcn_degree",
    )
    def deg_kernel(dst_hbm, w_hbm, out_hbm, db, wb, vals, acc_sh,
                   dsem, wsem, scsem):
        c = lax.axis_index("c")
        s = lax.axis_index("s")
        wid = c * _NS + s
        base = wid * epw

        def zrow(i, carry):
            vals[0][i, pl.ds(0, _L)] = jnp.zeros((_L,), jnp.float32)
            return carry

        lax.fori_loop(0, batch, zrow, 0)

        def zero_stripe(r0, nrows):
            nfull, tail = nrows // batch, nrows % batch
            for k in range(nfull):
                pltpu.sync_copy(vals[0],
                                acc_sh.at[pl.ds(r0 + k * batch, batch)])
            if tail:
                pltpu.sync_copy(vals[0].at[pl.ds(0, tail)],
                                acc_sh.at[pl.ds(r0 + nfull * batch, tail)])

        _per_stripe(s, n, zero_stripe)
        plsc.subcore_barrier()

        def load(j, p):
            b0 = base + j * batch
            pltpu.async_copy(dst_hbm.at[pl.ds(b0, batch)], db[p], dsem[p])
            pltpu.async_copy(w_hbm.at[pl.ds(b0, batch)], wb[p], wsem[p])

        def wait_load(p):
            pltpu.make_async_copy(dst_hbm.at[pl.ds(base, batch)],
                                  db[p], dsem[p]).wait()
            pltpu.make_async_copy(w_hbm.at[pl.ds(base, batch)],
                                  wb[p], wsem[p]).wait()

        def work(j, p):
            def group_body(g, carry2):
                w_vec = wb[p][pl.ds(g * _L, _L)]
                for i in range(_L):
                    we = w_vec[i]
                    vals[p][g * _L + i, pl.ds(0, _L)] = (
                        jnp.ones((_L,), jnp.float32) * we)
                return carry2

            lax.fori_loop(0, batch // _L, group_body, 0)

        def scat(j, p):
            pltpu.async_copy(vals[p], acc_sh.at[db[p]], scsem[p], add=True)

        def wait_scat(p):
            pltpu.make_async_copy(vals[p], acc_sh.at[db[p]],
                                  scsem[p]).wait()

        _pipeline2(nb, load, wait_load, work, scat, wait_scat)
        plsc.subcore_barrier()

        def write_stripe(r0, rows):
            pltpu.sync_copy(acc_sh.at[pl.ds(r0, rows)],
                            out_hbm.at[pl.ds(c * n + r0, rows)])

        _per_stripe(s, n, write_stripe)

    return deg_kernel(dst, w)


def _sc_aggregate(hs, src, dst, w, *, n, d, e, batch):
    epw = e // _NW
    nb = epw // batch

    @functools.partial(
        pl.kernel,
        out_type=jax.ShapeDtypeStruct((_NC * n, d), jnp.float32),
        mesh=_sc_mesh(),
        scratch_types=[
            pltpu.VMEM((epw,), jnp.int32),
            pltpu.VMEM((epw,), jnp.int32),
            pltpu.VMEM((epw,), jnp.float32),
            [pltpu.VMEM((batch, d), jnp.float32)] * 2,
            pltpu.VMEM_SHARED((n, d), jnp.float32),
            [pltpu.SemaphoreType.DMA] * 2,
            [pltpu.SemaphoreType.DMA] * 2,
        ],
        name="sc_gcn_aggregate",
    )
    def agg_kernel(hs_hbm, src_hbm, dst_hbm, w_hbm, out_hbm,
                   src_v, dst_v, w_v, rows, acc_sh, gsem, scsem):
        c = lax.axis_index("c")
        s = lax.axis_index("s")
        wid = c * _NS + s
        base = wid * epw
        pltpu.sync_copy(src_hbm.at[pl.ds(base, epw)], src_v)
        pltpu.sync_copy(dst_hbm.at[pl.ds(base, epw)], dst_v)
        pltpu.sync_copy(w_hbm.at[pl.ds(base, epw)], w_v)

        def zrow(i, carry):
            for cc in range(d // _L):
                rows[0][i, pl.ds(cc * _L, _L)] = jnp.zeros((_L,), jnp.float32)
            return carry

        lax.fori_loop(0, batch, zrow, 0)

        def zero_stripe(r0, nrows):
            nfull, tail = nrows // batch, nrows % batch
            for k in range(nfull):
                pltpu.sync_copy(rows[0],
                                acc_sh.at[pl.ds(r0 + k * batch, batch)])
            if tail:
                pltpu.sync_copy(rows[0].at[pl.ds(0, tail)],
                                acc_sh.at[pl.ds(r0 + nfull * batch, tail)])

        _per_stripe(s, n, zero_stripe)
        plsc.subcore_barrier()

        def load(j, p):
            pltpu.async_copy(hs_hbm.at[src_v.at[pl.ds(j * batch, batch)]],
                             rows[p], gsem[p])

        def wait_load(p):
            pltpu.make_async_copy(hs_hbm.at[src_v.at[pl.ds(0, batch)]],
                                  rows[p], gsem[p]).wait()

        def work(j, p):
            buf = rows[p]

            def group_body(g, carry):
                w_vec = w_v[pl.ds(j * batch + g * _L, _L)]
                for i in range(_L):
                    we = w_vec[i]
                    row = g * _L + i
                    for cc in range(d // _L):
                        sl = pl.ds(cc * _L, _L)
                        buf[row, sl] = buf[row, sl] * we
                return carry

            lax.fori_loop(0, batch // _L, group_body, 0)

        def scat(j, p):
            pltpu.async_copy(rows[p],
                             acc_sh.at[dst_v.at[pl.ds(j * batch, batch)]],
                             scsem[p], add=True)

        def wait_scat(p):
            pltpu.make_async_copy(rows[p],
                                  acc_sh.at[dst_v.at[pl.ds(0, batch)]],
                                  scsem[p]).wait()

        _pipeline2(nb, load, wait_load, work, scat, wait_scat)
        plsc.subcore_barrier()

        def write_stripe(r0, rows):
            pltpu.sync_copy(acc_sh.at[pl.ds(r0, rows)],
                            out_hbm.at[pl.ds(c * n + r0, rows)])

        _per_stripe(s, n, write_stripe)

    return agg_kernel(hs, src, dst, w)


def _row_spec(bn, d, offset_blocks=0):
    return pl.BlockSpec((bn, d), lambda i, o=offset_blocks: (i + o, 0))


def _tc_layer1(x, w1, degp, *, n, d, bn):
    grid = (n // bn,)

    def body(x_ref, w_ref, d0_ref, d1_ref, out_ref):
        deg = d0_ref[:, 0:1] + d1_ref[:, 0:1] + 2.0
        dis = lax.rsqrt(deg)
        h = jnp.dot(x_ref[...], w_ref[...],
                    preferred_element_type=jnp.float32)
        out_ref[...] = h * dis

    return pl.pallas_call(
        body,
        grid=grid,
        in_specs=[
            _row_spec(bn, d),
            pl.BlockSpec((d, d), lambda i: (0, 0)),
            _row_spec(bn, 16),
            _row_spec(bn, 16, n // bn),
        ],
        out_specs=_row_spec(bn, d),
        out_shape=jax.ShapeDtypeStruct((n, d), jnp.float32),
    )(x, w1, degp, degp)


def _tc_layer2(aggp, hs, degp, b1, w2, *, n, d, bn):
    grid = (n // bn,)

    def body(a0_ref, a1_ref, hs_ref, d0_ref, d1_ref, b_ref, w_ref, out_ref):
        deg = d0_ref[:, 0:1] + d1_ref[:, 0:1] + 2.0
        dis = lax.rsqrt(deg)
        t = dis * (a0_ref[...] + a1_ref[...]) + (2.0 * dis) * hs_ref[...]
        z = jnp.maximum(t + b_ref[...], 0.0)
        h = jnp.dot(z, w_ref[...], preferred_element_type=jnp.float32)
        out_ref[...] = h * dis

    return pl.pallas_call(
        body,
        grid=grid,
        in_specs=[
            _row_spec(bn, d),
            _row_spec(bn, d, n // bn),
            _row_spec(bn, d),
            _row_spec(bn, 16),
            _row_spec(bn, 16, n // bn),
            pl.BlockSpec((1, d), lambda i: (0, 0)),
            pl.BlockSpec((d, d), lambda i: (0, 0)),
        ],
        out_specs=_row_spec(bn, d),
        out_shape=jax.ShapeDtypeStruct((n, d), jnp.float32),
    )(aggp, aggp, hs, degp, degp, b1, w2)


def _tc_final(aggp, hs, degp, b2, *, n, d, bn):
    grid = (n // bn,)

    def body(a0_ref, a1_ref, hs_ref, d0_ref, d1_ref, b_ref, out_ref):
        deg = d0_ref[:, 0:1] + d1_ref[:, 0:1] + 2.0
        dis = lax.rsqrt(deg)
        t = dis * (a0_ref[...] + a1_ref[...]) + (2.0 * dis) * hs_ref[...]
        out_ref[...] = jax.nn.sigmoid(t + b_ref[...])

    return pl.pallas_call(
        body,
        grid=grid,
        in_specs=[
            _row_spec(bn, d),
            _row_spec(bn, d, n // bn),
            _row_spec(bn, d),
            _row_spec(bn, 16),
            _row_spec(bn, 16, n // bn),
            pl.BlockSpec((1, d), lambda i: (0, 0)),
        ],
        out_specs=_row_spec(bn, d),
        out_shape=jax.ShapeDtypeStruct((n, d), jnp.float32),
    )(aggp, aggp, hs, degp, degp, b2)


def kernel(x, edges, weights, W1, b1, W2, b2):
    n, d = x.shape
    e = edges.shape[1]
    batch_a = 80
    batch_d = 128
    bn = 1000

    src = edges[0].astype(jnp.int32)
    dst = edges[1].astype(jnp.int32)
    w = weights.astype(jnp.float32)
    b1r = b1.reshape(1, d)
    b2r = b2.reshape(1, d)

    nb_d = -(-e // (_NW * batch_d)) | 1
    e_d = _NW * nb_d * batch_d
    pad_idx = (jnp.arange(e_d - e, dtype=jnp.int32) * 97) % n
    dst_d = jnp.concatenate([dst, pad_idx])
    w_d = jnp.concatenate([w, jnp.zeros((e_d - e,), jnp.float32)])

    degp = _sc_degree(dst_d, w_d, n=n, e=e_d, batch=batch_d)
    h1s = _tc_layer1(x, W1, degp, n=n, d=d, bn=bn)
    agg1 = _sc_aggregate(h1s, src, dst, w, n=n, d=d, e=e, batch=batch_a)
    h2s = _tc_layer2(agg1, h1s, degp, b1r, W2, n=n, d=d, bn=bn)
    agg2 = _sc_aggregate(h2s, src, dst, w, n=n, d=d, e=e, batch=batch_a)
    return _tc_final(agg2, h2s, degp, b2r, n=n, d=d, bn=bn)
